# Initial kernel scaffold; baseline (speedup 1.0000x reference)
import jax
import jax.numpy as jnp
from jax.experimental import pallas as pl


def kernel(x, edge_index, edge_weight, W):
    raise NotImplementedError("write your pallas kernel here")


# speedup vs baseline: 4.1035x; 4.1035x over previous
import functools

import jax
import jax.numpy as jnp
from jax import lax
from jax.experimental import pallas as pl
from jax.experimental.pallas import tpu as pltpu
from jax.experimental.pallas import tpu_sc as plsc

CHUNK = 128
LANES = 16


def _mm_body(x_ref, w_ref, o_ref):
    o_ref[...] = jnp.dot(x_ref[...], w_ref[...], preferred_element_type=jnp.float32)


def _combine_body(p_ref, o_ref):
    o_ref[...] = jnp.maximum(jnp.sum(p_ref[...], axis=0), 0.0)


def _edge_kernel(N, D, E_pad, NC, NS):
    NW = NC * NS
    n_chunks = E_pad // CHUNK
    steps = n_chunks // NW
    rows_per_tile = (N // NS) // 8 * 8
    tail_start = NS * rows_per_tile
    tail_rows = N - tail_start
    nsub = D // LANES

    mesh = plsc.VectorSubcoreMesh(
        core_axis_name="c", subcore_axis_name="s", num_cores=NC, num_subcores=NS
    )

    @functools.partial(
        pl.kernel,
        mesh=mesh,
        out_type=jax.ShapeDtypeStruct((NC, N, D), jnp.float32),
        scratch_types=[
            pltpu.VMEM_SHARED((N, D), jnp.float32),
            pltpu.VMEM((CHUNK,), jnp.int32),
            pltpu.VMEM((CHUNK,), jnp.int32),
            pltpu.VMEM((CHUNK,), jnp.float32),
            pltpu.VMEM((CHUNK, D), jnp.float32),
            pltpu.SemaphoreType.DMA,
        ],
    )
    def body(pre_hbm, src_hbm, dst_hbm, w_hbm, out_hbm, accum, sidx, didx, wbuf, rbuf, sem):
        cid = lax.axis_index("c")
        sid = lax.axis_index("s")
        wid = sid * NC + cid

        def zero_rbuf(i, _):
            for j in range(nsub):
                rbuf[i, pl.ds(j * LANES, LANES)] = jnp.zeros((LANES,), jnp.float32)
            return 0

        lax.fori_loop(0, CHUNK, zero_rbuf, 0)
        row0 = sid * rows_per_tile
        full, rem = divmod(rows_per_tile, CHUNK)
        for r in range(full):
            pltpu.sync_copy(rbuf, accum.at[pl.ds(row0 + r * CHUNK, CHUNK)])
        if rem:
            pltpu.sync_copy(
                rbuf.at[pl.ds(0, rem)], accum.at[pl.ds(row0 + full * CHUNK, rem)]
            )
        if tail_rows:

            @pl.when(sid == 0)
            def _():
                pltpu.sync_copy(
                    rbuf.at[pl.ds(0, tail_rows)], accum.at[pl.ds(tail_start, tail_rows)]
                )

        plsc.subcore_barrier()

        def edge_step(g, _):
            base = (g * NW + wid) * CHUNK
            pltpu.sync_copy(src_hbm.at[pl.ds(base, CHUNK)], sidx)
            pltpu.sync_copy(dst_hbm.at[pl.ds(base, CHUNK)], didx)
            pltpu.sync_copy(w_hbm.at[pl.ds(base, CHUNK)], wbuf)
            pltpu.async_copy(pre_hbm.at[sidx], rbuf, sem).wait()

            def scale_grp(g, _):
                wv16 = wbuf[pl.ds(g * LANES, LANES)]
                for k in range(LANES):
                    wk = wv16[k]
                    e = g * LANES + k
                    for j in range(nsub):
                        sl = pl.ds(j * LANES, LANES)
                        rbuf[e, sl] = rbuf[e, sl] * wk
                return 0

            lax.fori_loop(0, CHUNK // LANES, scale_grp, 0)
            pltpu.sync_copy(rbuf, accum.at[didx], add=True)
            return 0

        lax.fori_loop(0, steps, edge_step, 0)
        plsc.subcore_barrier()

        pltpu.sync_copy(
            accum.at[pl.ds(row0, rows_per_tile)],
            out_hbm.at[cid, pl.ds(row0, rows_per_tile)],
        )
        if tail_rows:

            @pl.when(sid == 0)
            def _():
                pltpu.sync_copy(
                    accum.at[pl.ds(tail_start, tail_rows)],
                    out_hbm.at[cid, pl.ds(tail_start, tail_rows)],
                )

    return body


def kernel(x, edge_index, edge_weight, W):
    N, D_in = x.shape
    D = W.shape[1]
    E = edge_weight.shape[0]

    info = plsc.get_sparse_core_info()
    NC, NS = info.num_cores, info.num_subcores
    NW = NC * NS

    RB = 1000
    assert N % RB == 0 and D % LANES == 0
    pre_sup = pl.pallas_call(
        _mm_body,
        grid=(N // RB,),
        in_specs=[
            pl.BlockSpec((RB, D_in), lambda i: (i, 0)),
            pl.BlockSpec((D_in, D), lambda i: (0, 0)),
        ],
        out_specs=pl.BlockSpec((RB, D), lambda i: (i, 0)),
        out_shape=jax.ShapeDtypeStruct((N, D), jnp.float32),
    )(x, W)

    src = edge_index[0]
    dst = edge_index[1]
    steps = -(-E // (CHUNK * NW))
    E_pad = steps * CHUNK * NW
    if E_pad != E:
        pad = E_pad - E
        src = jnp.concatenate([src, jnp.zeros((pad,), jnp.int32)])
        dst = jnp.concatenate([dst, jnp.zeros((pad,), jnp.int32)])
        edge_weight = jnp.concatenate([edge_weight, jnp.zeros((pad,), jnp.float32)])

    partial = _edge_kernel(N, D, E_pad, NC, NS)(pre_sup, src, dst, edge_weight)

    out = pl.pallas_call(
        _combine_body,
        grid=(N // RB,),
        in_specs=[pl.BlockSpec((NC, RB, D), lambda i: (0, i, 0))],
        out_specs=pl.BlockSpec((RB, D), lambda i: (i, 0)),
        out_shape=jax.ShapeDtypeStruct((N, D), jnp.float32),
    )(partial)
    return out
